# baseline (speedup 1.0000x reference)
import functools

import numpy as np

import jax
import jax.numpy as jnp
from jax.experimental import pallas as pl
from jax.experimental.pallas import tpu as pltpu
from jax.sharding import Mesh, PartitionSpec as P

_U32 = jnp.uint32
_BERN_THRESH = 7549747
_ROTS = ((13, 15, 26, 6), (17, 29, 16, 24))


def _threefry_bits(c_lo, k0, k1):
    ks2 = k0 ^ k1 ^ _U32(0x1BD11BDA)
    ks = (k0, k1, ks2)
    x0 = jnp.zeros_like(c_lo) + k0
    x1 = c_lo + k1
    for g in range(5):
        for r in _ROTS[g % 2]:
            x0 = x0 + x1
            x1 = ((x1 << _U32(r)) | (x1 >> _U32(32 - r))) ^ x0
        x0 = x0 + ks[(g + 1) % 3]
        x1 = x1 + ks[(g + 2) % 3] + _U32(g + 1)
    return x0 ^ x1


def _l1_kernel(key_ref, off_ref, adj_ref, sa_ref, sb_ref, b_ref,
               oa_ref, ob_ref, a16_ref, m_ref, *, bm, n):
    r0 = off_ref[0] + pl.program_id(0) * bm
    rows = jax.lax.broadcasted_iota(jnp.int32, (bm, n), 0) + r0
    cols = jax.lax.broadcasted_iota(jnp.int32, (bm, n), 1)
    c_lo = (rows * n + cols).astype(_U32)
    bits = _threefry_bits(c_lo, key_ref[0], key_ref[1])
    mask = (bits >> _U32(9)) < _U32(_BERN_THRESH)
    m_ref[...] = mask.astype(jnp.uint8)

    a = adj_ref[...].astype(jnp.bfloat16)
    a16_ref[...] = a
    av = jnp.where(mask, a, jnp.bfloat16(0.0))
    b = b_ref[...]
    dn = (((1,), (0,)), ((), ()))
    oa = jax.lax.dot_general(a, sa_ref[...], dn,
                             preferred_element_type=jnp.float32)
    ob = jax.lax.dot_general(av, sb_ref[...], dn,
                             preferred_element_type=jnp.float32)
    oa_ref[...] = jnp.maximum(oa + b, 0.0)
    ob_ref[...] = jnp.maximum(ob + b, 0.0)


def _layer1(key_words, row_off, adj, sa, sb, bias, bm):
    rows, n = adj.shape
    f = sa.shape[1]
    full = lambda i: (0, 0)
    blk = lambda i: (i, 0)
    return pl.pallas_call(
        functools.partial(_l1_kernel, bm=bm, n=n),
        grid=(rows // bm,),
        in_specs=[
            pl.BlockSpec(memory_space=pltpu.SMEM),
            pl.BlockSpec(memory_space=pltpu.SMEM),
            pl.BlockSpec((bm, n), blk),
            pl.BlockSpec((n, f), full),
            pl.BlockSpec((n, f), full),
            pl.BlockSpec((1, f), full),
        ],
        out_specs=[
            pl.BlockSpec((bm, f), blk),
            pl.BlockSpec((bm, f), blk),
            pl.BlockSpec((bm, n), blk),
            pl.BlockSpec((bm, n), blk),
        ],
        out_shape=[
            jax.ShapeDtypeStruct((rows, f), jnp.float32),
            jax.ShapeDtypeStruct((rows, f), jnp.float32),
            jax.ShapeDtypeStruct((rows, n), jnp.bfloat16),
            jax.ShapeDtypeStruct((rows, n), jnp.uint8),
        ],
    )(key_words, row_off, adj, sa, sb, bias)


def _l2_kernel(a16_ref, m_ref, sa_ref, sb_ref, b_ref, oa_ref, ob_ref):
    a = a16_ref[...]
    av = jnp.where(m_ref[...] != 0, a, jnp.bfloat16(0.0))
    b = b_ref[...]
    dn = (((1,), (0,)), ((), ()))
    oa = jax.lax.dot_general(a, sa_ref[...], dn,
                             preferred_element_type=jnp.float32)
    ob = jax.lax.dot_general(av, sb_ref[...], dn,
                             preferred_element_type=jnp.float32)
    oa_ref[...] = jnp.maximum(oa + b, 0.0)
    ob_ref[...] = jnp.maximum(ob + b, 0.0)


def _layer2(a16, mask, sa, sb, bias, bm):
    rows, n = a16.shape
    f = sa.shape[1]
    full = lambda i: (0, 0)
    blk = lambda i: (i, 0)
    return pl.pallas_call(
        _l2_kernel,
        grid=(rows // bm,),
        in_specs=[
            pl.BlockSpec((bm, n), blk),
            pl.BlockSpec((bm, n), blk),
            pl.BlockSpec((n, f), full),
            pl.BlockSpec((n, f), full),
            pl.BlockSpec((1, f), full),
        ],
        out_specs=[
            pl.BlockSpec((bm, f), blk),
            pl.BlockSpec((bm, f), blk),
        ],
        out_shape=[
            jax.ShapeDtypeStruct((rows, f), jnp.float32),
            jax.ShapeDtypeStruct((rows, f), jnp.float32),
        ],
    )(a16, mask, sa, sb, bias)


def _matmul2w_kernel(x_ref, wa_ref, wb_ref, oa_ref, ob_ref):
    x = x_ref[...].astype(jnp.bfloat16)
    wa = wa_ref[...].astype(jnp.bfloat16)
    wb = wb_ref[...].astype(jnp.bfloat16)
    oa_ref[...] = jnp.dot(x, wa, preferred_element_type=jnp.float32).astype(
        jnp.bfloat16)
    ob_ref[...] = jnp.dot(x, wb, preferred_element_type=jnp.float32).astype(
        jnp.bfloat16)


def _matmul2_kernel(xa_ref, xb_ref, w_ref, oa_ref, ob_ref):
    w = w_ref[...].astype(jnp.bfloat16)
    xa = xa_ref[...].astype(jnp.bfloat16)
    xb = xb_ref[...].astype(jnp.bfloat16)
    oa_ref[...] = jnp.dot(xa, w, preferred_element_type=jnp.float32).astype(
        jnp.bfloat16)
    ob_ref[...] = jnp.dot(xb, w, preferred_element_type=jnp.float32).astype(
        jnp.bfloat16)


def _matmul2w(x, wa, wb):
    n = x.shape[0]
    f = wa.shape[1]
    return pl.pallas_call(
        _matmul2w_kernel,
        out_shape=[
            jax.ShapeDtypeStruct((n, f), jnp.bfloat16),
            jax.ShapeDtypeStruct((n, f), jnp.bfloat16),
        ],
    )(x, wa, wb)


def _matmul2(xa, xb, w):
    n = xa.shape[0]
    f = w.shape[1]
    return pl.pallas_call(
        _matmul2_kernel,
        out_shape=[
            jax.ShapeDtypeStruct((n, f), jnp.bfloat16),
            jax.ShapeDtypeStruct((n, f), jnp.bfloat16),
        ],
    )(xa, xb, w)


def kernel(x, adj, W0, b0, W1, b1, sparse=0):
    n = adj.shape[0]
    devs = jax.devices()
    m = len(devs)
    while m > 1 and (n % m != 0 or (n // m) % 400 != 0):
        m -= 1
    mesh = Mesh(np.array(devs[:m]), ("i",))
    local_rows = n // m

    k1, k2 = jax.random.split(jax.random.key(1))
    key_words = jax.random.key_data(k1).astype(jnp.uint32)
    feat_mask = jax.random.uniform(k2, (x.shape[1],)) < 0.1
    W0m = jnp.where(feat_mask[:, None], 0.0, W0)
    b0r = b0.reshape(1, -1)
    b1r = b1.reshape(1, -1)

    def body(adj_l, x_r, w0m_r, w0_r, w1_r, b0_r, b1_r, kw_r):
        if m == 1:
            row_off = jnp.zeros((1,), jnp.int32)
        else:
            row_off = (jax.lax.axis_index("i").astype(jnp.int32)
                       * jnp.int32(local_rows)).reshape((1,))
        s0a, s0b = _matmul2w(x_r, w0m_r, w0_r)
        h1a_l, h1b_l, a16_l, mask_l = _layer1(
            key_words=kw_r, row_off=row_off, adj=adj_l,
            sa=s0a, sb=s0b, bias=b0_r, bm=80)
        s1a_l, s1b_l = _matmul2(h1a_l, h1b_l, w1_r)
        if m == 1:
            s1a, s1b = s1a_l, s1b_l
        else:
            s1a = jax.lax.all_gather(s1a_l, "i", axis=0, tiled=True)
            s1b = jax.lax.all_gather(s1b_l, "i", axis=0, tiled=True)
        h2a_l, h2b_l = _layer2(a16_l, mask_l, s1a, s1b, b1_r, bm=400)
        return h2a_l, h2b_l

    if m == 1:
        h2a, h2b = body(adj, x, W0m, W0, W1, b0r, b1r, key_words)
    else:
        rep = P(None, None)
        h2a, h2b = jax.shard_map(
            body, mesh=mesh,
            in_specs=(P("i", None), rep, rep, rep, rep, rep, rep, P(None)),
            out_specs=(P("i", None), P("i", None)),
            check_vma=False,
        )(adj, x, W0m, W0, W1, b0r, b1r, key_words)
    return (h2a, h2b)


# speedup vs baseline: 1.0004x; 1.0004x over previous
import functools

import numpy as np

import jax
import jax.numpy as jnp
from jax.experimental import pallas as pl
from jax.experimental.pallas import tpu as pltpu
from jax.sharding import Mesh, PartitionSpec as P

_U32 = jnp.uint32
_BERN_THRESH = 7549747
_ROTS = ((13, 15, 26, 6), (17, 29, 16, 24))


def _threefry_bits(c_lo, k0, k1):
    ks2 = k0 ^ k1 ^ _U32(0x1BD11BDA)
    ks = (k0, k1, ks2)
    x0 = jnp.zeros_like(c_lo) + k0
    x1 = c_lo + k1
    for g in range(5):
        for r in _ROTS[g % 2]:
            x0 = x0 + x1
            x1 = ((x1 << _U32(r)) | (x1 >> _U32(32 - r))) ^ x0
        x0 = x0 + ks[(g + 1) % 3]
        x1 = x1 + ks[(g + 2) % 3] + _U32(g + 1)
    return x0 ^ x1


def _l1_kernel(key_ref, off_ref, adj_ref, sa_ref, sb_ref, b_ref,
               oa_ref, ob_ref, a16_ref, m_ref, *, bm, n):
    r0 = off_ref[0] + pl.program_id(0) * bm
    rows = jax.lax.broadcasted_iota(jnp.int32, (bm, n), 0) + r0
    cols = jax.lax.broadcasted_iota(jnp.int32, (bm, n), 1)
    c_lo = (rows * n + cols).astype(_U32)
    bits = _threefry_bits(c_lo, key_ref[0], key_ref[1])
    mask = (bits >> _U32(9)) < _U32(_BERN_THRESH)
    m_ref[...] = mask.astype(jnp.uint8)

    a = adj_ref[...].astype(jnp.bfloat16)
    a16_ref[...] = a
    av = jnp.where(mask, a, jnp.bfloat16(0.0))
    b = b_ref[...]
    dn = (((1,), (0,)), ((), ()))
    oa = jax.lax.dot_general(a, sa_ref[...], dn,
                             preferred_element_type=jnp.float32)
    ob = jax.lax.dot_general(av, sb_ref[...], dn,
                             preferred_element_type=jnp.float32)
    oa_ref[...] = jnp.maximum(oa + b, 0.0)
    ob_ref[...] = jnp.maximum(ob + b, 0.0)


def _layer1(key_words, row_off, adj, sa, sb, bias, bm):
    rows, n = adj.shape
    f = sa.shape[1]
    full = lambda i: (0, 0)
    blk = lambda i: (i, 0)
    return pl.pallas_call(
        functools.partial(_l1_kernel, bm=bm, n=n),
        grid=(rows // bm,),
        in_specs=[
            pl.BlockSpec(memory_space=pltpu.SMEM),
            pl.BlockSpec(memory_space=pltpu.SMEM),
            pl.BlockSpec((bm, n), blk),
            pl.BlockSpec((n, f), full),
            pl.BlockSpec((n, f), full),
            pl.BlockSpec((1, f), full),
        ],
        out_specs=[
            pl.BlockSpec((bm, f), blk),
            pl.BlockSpec((bm, f), blk),
            pl.BlockSpec((bm, n), blk),
            pl.BlockSpec((bm, n), blk),
        ],
        out_shape=[
            jax.ShapeDtypeStruct((rows, f), jnp.float32),
            jax.ShapeDtypeStruct((rows, f), jnp.float32),
            jax.ShapeDtypeStruct((rows, n), jnp.bfloat16),
            jax.ShapeDtypeStruct((rows, n), jnp.uint8),
        ],
    )(key_words, row_off, adj, sa, sb, bias)


def _l2_kernel(a16_ref, m_ref, sa_ref, sb_ref, b_ref, oa_ref, ob_ref):
    a = a16_ref[...]
    av = jnp.where(m_ref[...] != 0, a, jnp.bfloat16(0.0))
    b = b_ref[...]
    dn = (((1,), (0,)), ((), ()))
    oa = jax.lax.dot_general(a, sa_ref[...], dn,
                             preferred_element_type=jnp.float32)
    ob = jax.lax.dot_general(av, sb_ref[...], dn,
                             preferred_element_type=jnp.float32)
    oa_ref[...] = jnp.maximum(oa + b, 0.0)
    ob_ref[...] = jnp.maximum(ob + b, 0.0)


def _layer2(a16, mask, sa, sb, bias, bm):
    rows, n = a16.shape
    f = sa.shape[1]
    full = lambda i: (0, 0)
    blk = lambda i: (i, 0)
    return pl.pallas_call(
        _l2_kernel,
        grid=(rows // bm,),
        in_specs=[
            pl.BlockSpec((bm, n), blk),
            pl.BlockSpec((bm, n), blk),
            pl.BlockSpec((n, f), full),
            pl.BlockSpec((n, f), full),
            pl.BlockSpec((1, f), full),
        ],
        out_specs=[
            pl.BlockSpec((bm, f), blk),
            pl.BlockSpec((bm, f), blk),
        ],
        out_shape=[
            jax.ShapeDtypeStruct((rows, f), jnp.float32),
            jax.ShapeDtypeStruct((rows, f), jnp.float32),
        ],
    )(a16, mask, sa, sb, bias)


def _matmul2w_kernel(x_ref, wa_ref, wb_ref, oa_ref, ob_ref):
    x = x_ref[...].astype(jnp.bfloat16)
    wa = wa_ref[...].astype(jnp.bfloat16)
    wb = wb_ref[...].astype(jnp.bfloat16)
    oa_ref[...] = jnp.dot(x, wa, preferred_element_type=jnp.float32).astype(
        jnp.bfloat16)
    ob_ref[...] = jnp.dot(x, wb, preferred_element_type=jnp.float32).astype(
        jnp.bfloat16)


def _matmul2_kernel(xa_ref, xb_ref, w_ref, oa_ref, ob_ref):
    w = w_ref[...].astype(jnp.bfloat16)
    xa = xa_ref[...].astype(jnp.bfloat16)
    xb = xb_ref[...].astype(jnp.bfloat16)
    oa_ref[...] = jnp.dot(xa, w, preferred_element_type=jnp.float32).astype(
        jnp.bfloat16)
    ob_ref[...] = jnp.dot(xb, w, preferred_element_type=jnp.float32).astype(
        jnp.bfloat16)


def _matmul2w(x, wa, wb):
    n = x.shape[0]
    f = wa.shape[1]
    return pl.pallas_call(
        _matmul2w_kernel,
        out_shape=[
            jax.ShapeDtypeStruct((n, f), jnp.bfloat16),
            jax.ShapeDtypeStruct((n, f), jnp.bfloat16),
        ],
    )(x, wa, wb)


def _matmul2(xa, xb, w):
    n = xa.shape[0]
    f = w.shape[1]
    return pl.pallas_call(
        _matmul2_kernel,
        out_shape=[
            jax.ShapeDtypeStruct((n, f), jnp.bfloat16),
            jax.ShapeDtypeStruct((n, f), jnp.bfloat16),
        ],
    )(xa, xb, w)


def kernel(x, adj, W0, b0, W1, b1, sparse=0):
    n = adj.shape[0]
    devs = jax.devices()
    m = len(devs)
    while m > 1 and (n % m != 0 or (n // m) % 80 != 0):
        m -= 1
    mesh = Mesh(np.array(devs[:m]), ("i",))
    local_rows = n // m

    k1, k2 = jax.random.split(jax.random.key(1))
    key_words = jax.random.key_data(k1).astype(jnp.uint32)
    feat_mask = jax.random.uniform(k2, (x.shape[1],)) < 0.1
    W0m = jnp.where(feat_mask[:, None], 0.0, W0)
    b0r = b0.reshape(1, -1)
    b1r = b1.reshape(1, -1)

    def body(adj_l, x_r, w0m_r, w0_r, w1_r, b0_r, b1_r, kw_r):
        if m == 1:
            row_off = jnp.zeros((1,), jnp.int32)
        else:
            row_off = (jax.lax.axis_index("i").astype(jnp.int32)
                       * jnp.int32(local_rows)).reshape((1,))
        s0a, s0b = _matmul2w(x_r, w0m_r, w0_r)
        h1a_l, h1b_l, a16_l, mask_l = _layer1(
            key_words=kw_r, row_off=row_off, adj=adj_l,
            sa=s0a, sb=s0b, bias=b0_r, bm=80)
        s1a_l, s1b_l = _matmul2(h1a_l, h1b_l, w1_r)
        if m == 1:
            s1a, s1b = s1a_l, s1b_l
        else:
            s1a = jax.lax.all_gather(s1a_l, "i", axis=0, tiled=True)
            s1b = jax.lax.all_gather(s1b_l, "i", axis=0, tiled=True)
        bm2 = 400 if local_rows % 400 == 0 else 200
        h2a_l, h2b_l = _layer2(a16_l, mask_l, s1a, s1b, b1_r, bm=bm2)
        return h2a_l, h2b_l

    if m == 1:
        h2a, h2b = body(adj, x, W0m, W0, W1, b0r, b1r, key_words)
    else:
        rep = P(None, None)
        h2a, h2b = jax.shard_map(
            body, mesh=mesh,
            in_specs=(P("i", None), rep, rep, rep, rep, rep, rep, P(None)),
            out_specs=(P("i", None), P("i", None)),
            check_vma=False,
        )(adj, x, W0m, W0, W1, b0r, b1r, key_words)
    return (h2a, h2b)


# speedup vs baseline: 1.1136x; 1.1131x over previous
import functools

import numpy as np

import jax
import jax.numpy as jnp
from jax.experimental import pallas as pl
from jax.experimental.pallas import tpu as pltpu
from jax.sharding import Mesh, PartitionSpec as P

_U32 = jnp.uint32
_BERN_THRESH = 7549747
_ROTS = ((13, 15, 26, 6), (17, 29, 16, 24))


def _threefry_bits(c_lo, k0, k1):
    ks2 = k0 ^ k1 ^ _U32(0x1BD11BDA)
    ks = (k0, k1, ks2)
    x0 = jnp.zeros_like(c_lo) + k0
    x1 = c_lo + k1
    for g in range(5):
        for r in _ROTS[g % 2]:
            x0 = x0 + x1
            x1 = ((x1 << _U32(r)) | (x1 >> _U32(32 - r))) ^ x0
        x0 = x0 + ks[(g + 1) % 3]
        x1 = x1 + ks[(g + 2) % 3] + _U32(g + 1)
    return x0 ^ x1


def _l1_kernel(key_ref, off_ref, adj_ref, sa_ref, sb_ref, b_ref,
               oa_ref, ob_ref, a16_ref, m_ref, *, bm, n):
    r0 = off_ref[0] + pl.program_id(0) * bm
    rows = jax.lax.broadcasted_iota(jnp.int32, (bm, n), 0) + r0
    cols = jax.lax.broadcasted_iota(jnp.int32, (bm, n), 1)
    c_lo = (rows * n + cols).astype(_U32)
    bits = _threefry_bits(c_lo, key_ref[0], key_ref[1])
    mask = (bits >> _U32(9)) < _U32(_BERN_THRESH)
    m_ref[...] = mask.astype(jnp.uint8)

    a = adj_ref[...].astype(jnp.bfloat16)
    a16_ref[...] = a
    av = jnp.where(mask, a, jnp.bfloat16(0.0))
    b = b_ref[...]
    dn = (((1,), (0,)), ((), ()))
    oa = jax.lax.dot_general(a, sa_ref[...], dn,
                             preferred_element_type=jnp.float32)
    ob = jax.lax.dot_general(av, sb_ref[...], dn,
                             preferred_element_type=jnp.float32)
    oa_ref[...] = jnp.maximum(oa + b, 0.0)
    ob_ref[...] = jnp.maximum(ob + b, 0.0)


def _layer1(key_words, row_off, adj, sa, sb, bias, bm):
    rows, n = adj.shape
    f = sa.shape[1]
    full = lambda i: (0, 0)
    blk = lambda i: (i, 0)
    return pl.pallas_call(
        functools.partial(_l1_kernel, bm=bm, n=n),
        grid=(rows // bm,),
        in_specs=[
            pl.BlockSpec(memory_space=pltpu.SMEM),
            pl.BlockSpec(memory_space=pltpu.SMEM),
            pl.BlockSpec((bm, n), blk),
            pl.BlockSpec((n, f), full),
            pl.BlockSpec((n, f), full),
            pl.BlockSpec((1, f), full),
        ],
        out_specs=[
            pl.BlockSpec((bm, f), blk),
            pl.BlockSpec((bm, f), blk),
            pl.BlockSpec((bm, n), blk),
            pl.BlockSpec((bm, n), blk),
        ],
        out_shape=[
            jax.ShapeDtypeStruct((rows, f), jnp.float32),
            jax.ShapeDtypeStruct((rows, f), jnp.float32),
            jax.ShapeDtypeStruct((rows, n), jnp.bfloat16),
            jax.ShapeDtypeStruct((rows, n), jnp.uint8),
        ],
    )(key_words, row_off, adj, sa, sb, bias)


def _l2_kernel(a16_ref, m_ref, sa_ref, sb_ref, b_ref, oa_ref, ob_ref):
    a = a16_ref[...]
    av = jnp.where(m_ref[...] != 0, a, jnp.bfloat16(0.0))
    b = b_ref[...]
    dn = (((1,), (0,)), ((), ()))
    oa = jax.lax.dot_general(a, sa_ref[...], dn,
                             preferred_element_type=jnp.float32)
    ob = jax.lax.dot_general(av, sb_ref[...], dn,
                             preferred_element_type=jnp.float32)
    oa_ref[...] = jnp.maximum(oa + b, 0.0)
    ob_ref[...] = jnp.maximum(ob + b, 0.0)


def _layer2(a16, mask, sa, sb, bias, bm):
    rows, n = a16.shape
    f = sa.shape[1]
    full = lambda i: (0, 0)
    blk = lambda i: (i, 0)
    return pl.pallas_call(
        _l2_kernel,
        grid=(rows // bm,),
        in_specs=[
            pl.BlockSpec((bm, n), blk),
            pl.BlockSpec((bm, n), blk),
            pl.BlockSpec((n, f), full),
            pl.BlockSpec((n, f), full),
            pl.BlockSpec((1, f), full),
        ],
        out_specs=[
            pl.BlockSpec((bm, f), blk),
            pl.BlockSpec((bm, f), blk),
        ],
        out_shape=[
            jax.ShapeDtypeStruct((rows, f), jnp.float32),
            jax.ShapeDtypeStruct((rows, f), jnp.float32),
        ],
    )(a16, mask, sa, sb, bias)


def _matmul2w_kernel(x_ref, wa_ref, wb_ref, oa_ref, ob_ref):
    x = x_ref[...].astype(jnp.bfloat16)
    wa = wa_ref[...].astype(jnp.bfloat16)
    wb = wb_ref[...].astype(jnp.bfloat16)
    oa_ref[...] = jnp.dot(x, wa, preferred_element_type=jnp.float32).astype(
        jnp.bfloat16)
    ob_ref[...] = jnp.dot(x, wb, preferred_element_type=jnp.float32).astype(
        jnp.bfloat16)


def _matmul2_kernel(xa_ref, xb_ref, w_ref, oa_ref, ob_ref):
    w = w_ref[...].astype(jnp.bfloat16)
    xa = xa_ref[...].astype(jnp.bfloat16)
    xb = xb_ref[...].astype(jnp.bfloat16)
    oa_ref[...] = jnp.dot(xa, w, preferred_element_type=jnp.float32).astype(
        jnp.bfloat16)
    ob_ref[...] = jnp.dot(xb, w, preferred_element_type=jnp.float32).astype(
        jnp.bfloat16)


def _matmul2w(x, wa, wb):
    n = x.shape[0]
    f = wa.shape[1]
    return pl.pallas_call(
        _matmul2w_kernel,
        out_shape=[
            jax.ShapeDtypeStruct((n, f), jnp.bfloat16),
            jax.ShapeDtypeStruct((n, f), jnp.bfloat16),
        ],
    )(x, wa, wb)


def _matmul2(xa, xb, w):
    n = xa.shape[0]
    f = w.shape[1]
    return pl.pallas_call(
        _matmul2_kernel,
        out_shape=[
            jax.ShapeDtypeStruct((n, f), jnp.bfloat16),
            jax.ShapeDtypeStruct((n, f), jnp.bfloat16),
        ],
    )(xa, xb, w)


def kernel(x, adj, W0, b0, W1, b1, sparse=0):
    n = adj.shape[0]
    devs = jax.devices()
    m = len(devs)
    while m > 1 and (n % m != 0 or (n // m) % 40 != 0):
        m -= 1
    mesh = Mesh(np.array(devs[:m]), ("i",))
    local_rows = n // m

    k1, k2 = jax.random.split(jax.random.key(1))
    key_words = jax.random.key_data(k1).astype(jnp.uint32)
    feat_mask = jax.random.uniform(k2, (x.shape[1],)) < 0.1
    W0m = jnp.where(feat_mask[:, None], 0.0, W0)
    b0r = b0.reshape(1, -1)
    b1r = b1.reshape(1, -1)

    def body(adj_l, x_r, w0m_r, w0_r, w1_r, b0_r, b1_r, kw_r):
        if m == 1:
            row_off = jnp.zeros((1,), jnp.int32)
        else:
            row_off = (jax.lax.axis_index("i").astype(jnp.int32)
                       * jnp.int32(local_rows)).reshape((1,))
        s0a, s0b = _matmul2w(x_r, w0m_r, w0_r)
        bm1 = 80 if local_rows % 80 == 0 else 40
        h1a_l, h1b_l, a16_l, mask_l = _layer1(
            key_words=kw_r, row_off=row_off, adj=adj_l,
            sa=s0a, sb=s0b, bias=b0_r, bm=bm1)
        s1a_l, s1b_l = _matmul2(h1a_l, h1b_l, w1_r)
        if m == 1:
            s1a, s1b = s1a_l, s1b_l
        else:
            s1a = jax.lax.all_gather(s1a_l, "i", axis=0, tiled=True)
            s1b = jax.lax.all_gather(s1b_l, "i", axis=0, tiled=True)
        bm2 = 400 if local_rows % 400 == 0 else 200
        h2a_l, h2b_l = _layer2(a16_l, mask_l, s1a, s1b, b1_r, bm=bm2)
        return h2a_l, h2b_l

    if m == 1:
        h2a, h2b = body(adj, x, W0m, W0, W1, b0r, b1r, key_words)
    else:
        rep = P(None, None)
        h2a, h2b = jax.shard_map(
            body, mesh=mesh,
            in_specs=(P("i", None), rep, rep, rep, rep, rep, rep, P(None)),
            out_specs=(P("i", None), P("i", None)),
            check_vma=False,
        )(adj, x, W0m, W0, W1, b0r, b1r, key_words)
    return (h2a, h2b)
